# baseline (speedup 1.0000x reference)
import functools

import jax
import jax.numpy as jnp
from jax.experimental import pallas as pl
from jax.experimental.pallas import tpu as pltpu

_BK = 4


def _gnn_kernel(B, N, HID, OUT,
                x_ref, adj_ref, Q_ref, W2c_ref,
                b2_ref, g1w_ref, g1b_ref, g2w_ref, g2b_ref,
                out_ref,
                sRows, adjs, sm1, sq1, sg1, sk1, h2s,
                sm2, sq2, sg2, sk2):
    p = pl.program_id(0)
    b = pl.program_id(1)

    @pl.when(p == 0)
    def _phase0():
        u = Q_ref[0:1]
        v = Q_ref[1:2]
        w = Q_ref[2:3]
        Suu = jnp.sum(u * u)
        Svv = jnp.sum(v * v)
        Sww = jnp.sum(w * w)
        Suv = jnp.sum(u * v)
        Suw = jnp.sum(u * w)
        Svw = jnp.sum(v * w)
        Su = jnp.sum(u)
        Sv = jnp.sum(v)
        Sw = jnp.sum(w)

        tot_rowsum = None
        tot_rowsq = None
        for i in range(_BK):
            adj_b = adj_ref[i]
            xb = x_ref[i]
            xcat = jnp.concatenate([xb, jnp.ones_like(xb)], axis=1)
            mm = jnp.dot(adj_b, xcat, preferred_element_type=jnp.float32)
            inv_deg = 1.0 / jnp.maximum(mm[:, 1:2], 1.0)
            adjs[b * _BK + i] = adj_b * inv_deg

            mt = jnp.swapaxes(jnp.concatenate([mm, xb], axis=1), 0, 1)
            m1r = mt[0:1]
            degr = jnp.maximum(mt[1:2], 1.0)
            xr = mt[2:3]
            a = m1r / degr

            q = (a * a * Suu + xr * xr * Svv + Sww
                 + 2.0 * a * xr * Suv + 2.0 * a * Suw + 2.0 * xr * Svw)
            nrm = jnp.maximum(jnp.sqrt(q), 1e-12)
            invn = 1.0 / nrm
            Ar = a * invn
            Sr = xr * invn
            sRows[b * _BK + i] = jnp.concatenate([Ar, Sr, invn, invn],
                                                 axis=0)

            rowsum = Ar * Su + Sr * Sv + invn * Sw
            rowsq = q * invn * invn
            tot_rowsum = rowsum if i == 0 else tot_rowsum + rowsum
            tot_rowsq = rowsq if i == 0 else tot_rowsq + rowsq

        @pl.when(b == 0)
        def _():
            sm1[...] = tot_rowsum
            sq1[...] = tot_rowsq

        @pl.when(b != 0)
        def _():
            sm1[...] += tot_rowsum
            sq1[...] += tot_rowsq

    @pl.when(p == 1)
    def _phase1():
        @pl.when(b == 0)
        def _():
            mean = sm1[...] / (B * HID)
            ex2 = sq1[...] / (B * HID)
            var = ex2 - mean * mean
            inv = jax.lax.rsqrt(var + 1e-5)
            g = inv * g1w_ref[...]
            sg1[...] = g
            sk1[...] = g1b_ref[...] - mean * g

        g1 = sg1[...]
        k1 = sk1[...]

        tot_rowsum = None
        tot_rowsq = None
        for i in range(_BK):
            bb = b * _BK + i
            rows = sRows[bb]
            scaled = rows[0:3] * g1
            P = jnp.swapaxes(
                jnp.concatenate([scaled, k1], axis=0), 0, 1)
            hb = jnp.maximum(
                jnp.dot(P, Q_ref[...], preferred_element_type=jnp.float32),
                0.0)

            t = jnp.dot(adjs[bb], hb, preferred_element_type=jnp.float32)
            cat = jnp.concatenate([t, hb], axis=1)
            h2 = (jnp.dot(cat, W2c_ref[...], preferred_element_type=jnp.float32)
                  + b2_ref[...])

            rs = jnp.sum(h2, axis=1, keepdims=True)
            q2 = jnp.sum(h2 * h2, axis=1, keepdims=True)
            nrm2 = jnp.maximum(jnp.sqrt(q2), 1e-12)
            inv2 = 1.0 / nrm2
            h2s[bb] = (h2 * inv2).astype(jnp.bfloat16)

            rowsum = rs * inv2
            rowsq = q2 * inv2 * inv2
            tot_rowsum = rowsum if i == 0 else tot_rowsum + rowsum
            tot_rowsq = rowsq if i == 0 else tot_rowsq + rowsq

        @pl.when(b == 0)
        def _():
            sm2[...] = tot_rowsum
            sq2[...] = tot_rowsq

        @pl.when(b != 0)
        def _():
            sm2[...] += tot_rowsum
            sq2[...] += tot_rowsq

    @pl.when(p == 2)
    def _phase2():
        @pl.when(b == 0)
        def _():
            mean = sm2[...] / (B * OUT)
            ex2 = sq2[...] / (B * OUT)
            var = ex2 - mean * mean
            inv = jax.lax.rsqrt(var + 1e-5)
            g = inv * g2w_ref[...]
            sg2[...] = g
            sk2[...] = g2b_ref[...] - mean * g

        for i in range(_BK):
            out_ref[i] = jnp.maximum(
                h2s[b * _BK + i].astype(jnp.float32) * sg2[...] + sk2[...],
                0.0)


def kernel(x, adj, W1_rel, W1_root, b1, W2_rel, W2_root, b2,
           bn1_w, bn1_b, bn2_w, bn2_b):
    B, N, _ = x.shape
    HID = W1_rel.shape[1]
    OUT = W2_rel.shape[1]
    Bg = B // _BK

    u = W1_rel.reshape(1, HID).astype(jnp.float32)
    v = W1_root.reshape(1, HID).astype(jnp.float32)
    w = b1.reshape(1, HID).astype(jnp.float32)
    Q = jnp.concatenate([u, v, w, jnp.ones((1, HID), jnp.float32)], axis=0)
    W2c = jnp.concatenate([W2_rel.astype(jnp.float32),
                           W2_root.astype(jnp.float32)], axis=0)
    b2r = b2.reshape(1, OUT).astype(jnp.float32)
    g1w = bn1_w.reshape(1, N).astype(jnp.float32)
    g1b = bn1_b.reshape(1, N).astype(jnp.float32)
    g2w = bn2_w.reshape(N, 1).astype(jnp.float32)
    g2b = bn2_b.reshape(N, 1).astype(jnp.float32)

    grid = (3, Bg)

    def const_spec(shape):
        nd = len(shape)
        return pl.BlockSpec(shape, lambda p, b, _nd=nd: (0,) * _nd)

    in_specs = [
        pl.BlockSpec((_BK, N, 1),
                     lambda p, b: (jnp.where(p == 0, b, Bg - 1), 0, 0)),
        pl.BlockSpec((_BK, N, N),
                     lambda p, b: (jnp.where(p == 0, b, Bg - 1), 0, 0)),
        const_spec((4, HID)),
        const_spec((2 * HID, OUT)),
        const_spec((1, OUT)),
        const_spec((1, N)),
        const_spec((1, N)),
        const_spec((N, 1)),
        const_spec((N, 1)),
    ]
    out_spec = pl.BlockSpec((_BK, N, OUT),
                            lambda p, b: (jnp.where(p == 2, b, 0), 0, 0))

    scratch_shapes = [
        pltpu.VMEM((B, 4, N), jnp.float32),
        pltpu.VMEM((B, N, N), jnp.float32),
        pltpu.VMEM((1, N), jnp.float32),
        pltpu.VMEM((1, N), jnp.float32),
        pltpu.VMEM((1, N), jnp.float32),
        pltpu.VMEM((1, N), jnp.float32),
        pltpu.VMEM((B, N, OUT), jnp.bfloat16),
        pltpu.VMEM((N, 1), jnp.float32),
        pltpu.VMEM((N, 1), jnp.float32),
        pltpu.VMEM((N, 1), jnp.float32),
        pltpu.VMEM((N, 1), jnp.float32),
    ]

    fn = functools.partial(_gnn_kernel, B, N, HID, OUT)
    return pl.pallas_call(
        fn,
        grid=grid,
        in_specs=in_specs,
        out_specs=out_spec,
        out_shape=jax.ShapeDtypeStruct((B, N, OUT), jnp.float32),
        scratch_shapes=scratch_shapes,
        compiler_params=pltpu.CompilerParams(
            vmem_limit_bytes=100 * 1024 * 1024,
        ),
    )(x.astype(jnp.float32), adj.astype(jnp.float32), Q, W2c, b2r,
      g1w, g1b, g2w, g2b)


# speedup vs baseline: 1.0576x; 1.0576x over previous
import functools

import jax
import jax.numpy as jnp
from jax.experimental import pallas as pl
from jax.experimental.pallas import tpu as pltpu

_BK = 4


def _gnn_kernel(B, N, HID, OUT,
                x_ref, adj_ref, Q_ref, W2r_ref, W2o_ref,
                b2_ref, g1w_ref, g1b_ref, g2w_ref, g2b_ref,
                out_ref,
                sRows, adjs, sm1, sq1, sg1, sk1, h2s,
                sm2, sq2, sg2, sk2):
    p = pl.program_id(0)
    b = pl.program_id(1)

    @pl.when(p == 0)
    def _phase0():
        u = Q_ref[0:1]
        v = Q_ref[1:2]
        w = Q_ref[2:3]
        Suu = jnp.sum(u * u)
        Svv = jnp.sum(v * v)
        Sww = jnp.sum(w * w)
        Suv = jnp.sum(u * v)
        Suw = jnp.sum(u * w)
        Svw = jnp.sum(v * w)
        Su = jnp.sum(u)
        Sv = jnp.sum(v)
        Sw = jnp.sum(w)

        tot_rowsum = None
        tot_rowsq = None
        for i in range(_BK):
            adj_b = adj_ref[i]
            xb = x_ref[i]
            xcat = jnp.concatenate([xb, jnp.ones_like(xb)], axis=1)
            mm = jnp.dot(adj_b, xcat, preferred_element_type=jnp.float32)
            inv_deg = 1.0 / jnp.maximum(mm[:, 1:2], 1.0)
            adjs[b * _BK + i] = adj_b * inv_deg

            mt = jnp.swapaxes(jnp.concatenate([mm, xb], axis=1), 0, 1)
            m1r = mt[0:1]
            degr = jnp.maximum(mt[1:2], 1.0)
            xr = mt[2:3]
            a = m1r / degr

            q = (a * a * Suu + xr * xr * Svv + Sww
                 + 2.0 * a * xr * Suv + 2.0 * a * Suw + 2.0 * xr * Svw)
            nrm = jnp.maximum(jnp.sqrt(q), 1e-12)
            invn = 1.0 / nrm
            Ar = a * invn
            Sr = xr * invn
            sRows[b * _BK + i] = jnp.concatenate([Ar, Sr, invn, invn],
                                                 axis=0)

            rowsum = Ar * Su + Sr * Sv + invn * Sw
            rowsq = q * invn * invn
            tot_rowsum = rowsum if i == 0 else tot_rowsum + rowsum
            tot_rowsq = rowsq if i == 0 else tot_rowsq + rowsq

        @pl.when(b == 0)
        def _():
            sm1[...] = tot_rowsum
            sq1[...] = tot_rowsq

        @pl.when(b != 0)
        def _():
            sm1[...] += tot_rowsum
            sq1[...] += tot_rowsq

    @pl.when(p == 1)
    def _phase1():
        @pl.when(b == 0)
        def _():
            mean = sm1[...] / (B * HID)
            ex2 = sq1[...] / (B * HID)
            var = ex2 - mean * mean
            inv = jax.lax.rsqrt(var + 1e-5)
            g = inv * g1w_ref[...]
            sg1[...] = g
            sk1[...] = g1b_ref[...] - mean * g

        g1 = sg1[...]
        k1 = sk1[...]

        tot_rowsum = None
        tot_rowsq = None
        for i in range(_BK):
            bb = b * _BK + i
            rows = sRows[bb]
            scaled = rows[0:3] * g1
            P = jnp.swapaxes(
                jnp.concatenate([scaled, k1], axis=0), 0, 1)
            hb = jnp.maximum(
                jnp.dot(P, Q_ref[...], preferred_element_type=jnp.float32),
                0.0)

            t = jnp.dot(adjs[bb], hb, preferred_element_type=jnp.float32)
            h2 = (jnp.dot(t, W2r_ref[...], preferred_element_type=jnp.float32)
                  + jnp.dot(hb, W2o_ref[...],
                            preferred_element_type=jnp.float32)
                  + b2_ref[...])

            rs = jnp.sum(h2, axis=1, keepdims=True)
            q2 = jnp.sum(h2 * h2, axis=1, keepdims=True)
            nrm2 = jnp.maximum(jnp.sqrt(q2), 1e-12)
            inv2 = 1.0 / nrm2
            h2s[bb] = (h2 * inv2).astype(jnp.bfloat16)

            rowsum = rs * inv2
            rowsq = q2 * inv2 * inv2
            tot_rowsum = rowsum if i == 0 else tot_rowsum + rowsum
            tot_rowsq = rowsq if i == 0 else tot_rowsq + rowsq

        @pl.when(b == 0)
        def _():
            sm2[...] = tot_rowsum
            sq2[...] = tot_rowsq

        @pl.when(b != 0)
        def _():
            sm2[...] += tot_rowsum
            sq2[...] += tot_rowsq

    @pl.when(p == 2)
    def _phase2():
        @pl.when(b == 0)
        def _():
            mean = sm2[...] / (B * OUT)
            ex2 = sq2[...] / (B * OUT)
            var = ex2 - mean * mean
            inv = jax.lax.rsqrt(var + 1e-5)
            g = inv * g2w_ref[...]
            sg2[...] = g
            sk2[...] = g2b_ref[...] - mean * g

        for i in range(_BK):
            out_ref[i] = jnp.maximum(
                h2s[b * _BK + i].astype(jnp.float32) * sg2[...] + sk2[...],
                0.0)


def kernel(x, adj, W1_rel, W1_root, b1, W2_rel, W2_root, b2,
           bn1_w, bn1_b, bn2_w, bn2_b):
    B, N, _ = x.shape
    HID = W1_rel.shape[1]
    OUT = W2_rel.shape[1]
    Bg = B // _BK

    u = W1_rel.reshape(1, HID).astype(jnp.float32)
    v = W1_root.reshape(1, HID).astype(jnp.float32)
    w = b1.reshape(1, HID).astype(jnp.float32)
    Q = jnp.concatenate([u, v, w, jnp.ones((1, HID), jnp.float32)], axis=0)
    b2r = b2.reshape(1, OUT).astype(jnp.float32)
    g1w = bn1_w.reshape(1, N).astype(jnp.float32)
    g1b = bn1_b.reshape(1, N).astype(jnp.float32)
    g2w = bn2_w.reshape(N, 1).astype(jnp.float32)
    g2b = bn2_b.reshape(N, 1).astype(jnp.float32)

    grid = (3, Bg)

    def const_spec(shape):
        nd = len(shape)
        return pl.BlockSpec(shape, lambda p, b, _nd=nd: (0,) * _nd)

    in_specs = [
        pl.BlockSpec((_BK, N, 1),
                     lambda p, b: (jnp.where(p == 0, b, Bg - 1), 0, 0)),
        pl.BlockSpec((_BK, N, N),
                     lambda p, b: (jnp.where(p == 0, b, Bg - 1), 0, 0)),
        const_spec((4, HID)),
        const_spec((HID, OUT)),
        const_spec((HID, OUT)),
        const_spec((1, OUT)),
        const_spec((1, N)),
        const_spec((1, N)),
        const_spec((N, 1)),
        const_spec((N, 1)),
    ]
    out_spec = pl.BlockSpec((_BK, N, OUT),
                            lambda p, b: (jnp.where(p == 2, b, 0), 0, 0))

    scratch_shapes = [
        pltpu.VMEM((B, 4, N), jnp.float32),
        pltpu.VMEM((B, N, N), jnp.float32),
        pltpu.VMEM((1, N), jnp.float32),
        pltpu.VMEM((1, N), jnp.float32),
        pltpu.VMEM((1, N), jnp.float32),
        pltpu.VMEM((1, N), jnp.float32),
        pltpu.VMEM((B, N, OUT), jnp.bfloat16),
        pltpu.VMEM((N, 1), jnp.float32),
        pltpu.VMEM((N, 1), jnp.float32),
        pltpu.VMEM((N, 1), jnp.float32),
        pltpu.VMEM((N, 1), jnp.float32),
    ]

    fn = functools.partial(_gnn_kernel, B, N, HID, OUT)
    return pl.pallas_call(
        fn,
        grid=grid,
        in_specs=in_specs,
        out_specs=out_spec,
        out_shape=jax.ShapeDtypeStruct((B, N, OUT), jnp.float32),
        scratch_shapes=scratch_shapes,
        compiler_params=pltpu.CompilerParams(
            vmem_limit_bytes=100 * 1024 * 1024,
        ),
    )(x.astype(jnp.float32), adj.astype(jnp.float32), Q,
      W2_rel.astype(jnp.float32), W2_root.astype(jnp.float32), b2r,
      g1w, g1b, g2w, g2b)
